# baseline (speedup 1.0000x reference)
import functools

import jax
import jax.numpy as jnp
from jax import lax
from jax.experimental import pallas as pl
from jax.experimental.pallas import tpu as pltpu
from jax.experimental.pallas import tpu_sc as plsc

_KERNEL_NUM = 64
_SIZE = 1048576
_LANES = 16
_NSUB = 4
_NBUF = 2
_C_SC = _SIZE // 2
_BT = 4096


def _make_sc_kernel(cols):
    info = plsc.get_sparse_core_info()
    num_workers = info.num_cores * info.num_subcores
    chunk = cols // num_workers
    sub = chunk // _NSUB

    mesh = plsc.VectorSubcoreMesh(core_axis_name="c", subcore_axis_name="s")

    @functools.partial(
        pl.kernel,
        out_type=jax.ShapeDtypeStruct((1, _SIZE), jnp.float32),
        mesh=mesh,
        scratch_types=[
            pltpu.VMEM((_LANES,), jnp.float32),
            pltpu.VMEM((_NBUF, 2, sub), jnp.float32),
            pltpu.VMEM((_NBUF, sub), jnp.float32),
            [pltpu.SemaphoreType.DMA] * _NBUF,
            [pltpu.SemaphoreType.DMA] * _NBUF,
            [pltpu.SemaphoreType.DMA] * _NBUF,
        ],
    )
    def blend(lam_hbm, const_hbm, pivots_hbm, out_hbm, lam_v, ibuf, obuf,
              lsems, rsems, osems):
        del pivots_hbm
        wid = lax.axis_index("s") * info.num_cores + lax.axis_index("c")
        base = wid * chunk

        pltpu.sync_copy(lam_hbm, lam_v.at[pl.ds(0, 1)])
        lam_s = lam_v[...][0] * jnp.float32(0.99999)

        scaled = lam_s * jnp.float32(_KERNEL_NUM - 1)
        left = scaled.astype(jnp.int32)
        left = jnp.minimum(jnp.maximum(left, 0), _KERNEL_NUM - 2)
        dist = (left.astype(jnp.float32) + jnp.float32(1.0)) - scaled
        one_minus = jnp.float32(1.0) - dist

        def start_in(g, slot):
            col = pl.ds(base + g * sub, sub)
            pltpu.async_copy(const_hbm.at[left, col],
                             ibuf.at[slot, 0], lsems[slot])
            pltpu.async_copy(const_hbm.at[left + 1, col],
                             ibuf.at[slot, 1], rsems[slot])

        start_in(0, 0)

        for g in range(_NSUB):
            slot = g % _NBUF
            nxt = (g + 1) % _NBUF
            if g + 1 < _NSUB:
                start_in(g + 1, nxt)
            pltpu.make_async_copy(
                const_hbm.at[left, pl.ds(base, sub)],
                ibuf.at[slot, 0], lsems[slot]).wait()
            pltpu.make_async_copy(
                const_hbm.at[left + 1, pl.ds(base, sub)],
                ibuf.at[slot, 1], rsems[slot]).wait()
            if g >= _NBUF:
                pltpu.make_async_copy(
                    obuf.at[slot],
                    out_hbm.at[0, pl.ds(base, sub)], osems[slot]).wait()

            @plsc.parallel_loop(0, sub, step=_LANES, unroll=8)
            def _(i):
                sl = pl.ds(i, _LANES)
                obuf[slot, sl] = (ibuf[slot, 0, sl] * dist
                                  + ibuf[slot, 1, sl] * one_minus)

            pltpu.async_copy(
                obuf.at[slot],
                out_hbm.at[0, pl.ds(base + g * sub, sub)], osems[slot])

        for slot in range(_NBUF):
            pltpu.make_async_copy(
                obuf.at[slot],
                out_hbm.at[0, pl.ds(base, sub)], osems[slot]).wait()

    return blend


def _make_tc_kernel(col_lo):
    blk0 = col_lo // _BT
    grid = (_SIZE - col_lo) // _BT

    def body(lidx_ref, sc_ref, lam_ref, const_ref, out_ref, ibuf, sems):
        del sc_ref
        j = pl.program_id(0)
        left = lidx_ref[0]

        def start_in(jj, slot):
            col = pl.ds((blk0 + jj) * _BT, _BT)
            pltpu.make_async_copy(
                const_ref.at[pl.ds(left, 1), col],
                ibuf.at[slot, 0], sems.at[slot, 0]).start()
            pltpu.make_async_copy(
                const_ref.at[pl.ds(left + 1, 1), col],
                ibuf.at[slot, 1], sems.at[slot, 1]).start()

        @pl.when(j == 0)
        def _():
            start_in(0, 0)

        @pl.when(j + 1 < grid)
        def _():
            start_in(j + 1, (j + 1) % 2)

        slot = j % 2
        pltpu.make_async_copy(
            const_ref.at[pl.ds(left, 1), pl.ds(0, _BT)],
            ibuf.at[slot, 0], sems.at[slot, 0]).wait()
        pltpu.make_async_copy(
            const_ref.at[pl.ds(left, 1), pl.ds(0, _BT)],
            ibuf.at[slot, 1], sems.at[slot, 1]).wait()

        lam_ = lam_ref[0] * jnp.float32(0.99999)
        scaled = lam_ * jnp.float32(_KERNEL_NUM - 1)
        lf = scaled.astype(jnp.int32)
        lf = jnp.minimum(jnp.maximum(lf, 0), _KERNEL_NUM - 2)
        dist = (lf.astype(jnp.float32) + jnp.float32(1.0)) - scaled
        out_ref[...] = (ibuf[slot, 0] * dist
                        + ibuf[slot, 1] * (jnp.float32(1.0) - dist))

    grid_spec = pltpu.PrefetchScalarGridSpec(
        num_scalar_prefetch=1,
        grid=(grid,),
        in_specs=[
            pl.BlockSpec(memory_space=pl.MemorySpace.ANY),
            pl.BlockSpec(memory_space=pltpu.SMEM),
            pl.BlockSpec(memory_space=pl.MemorySpace.ANY),
        ],
        out_specs=pl.BlockSpec((1, _BT), lambda j, lidx: (0, blk0 + j)),
        scratch_shapes=[
            pltpu.VMEM((2, 2, 1, _BT), jnp.float32),
            pltpu.SemaphoreType.DMA((2, 2)),
        ],
    )
    return pl.pallas_call(
        body,
        grid_spec=grid_spec,
        out_shape=jax.ShapeDtypeStruct((1, _SIZE), jnp.float32),
        input_output_aliases={1: 0},
    )


_blend_tc_full = _make_tc_kernel(0)


def kernel(lam, const, pivots):
    del pivots
    z = jnp.zeros((1, _SIZE), jnp.float32)
    lidx = jnp.floor(lam * 0.99999 * (_KERNEL_NUM - 1)).astype(jnp.int32)
    return _blend_tc_full(lidx, z, lam, const)
